# baseline (speedup 1.0000x reference)
import jax
import jax.numpy as jnp
from jax import lax
from jax.experimental import pallas as pl
from jax.experimental.pallas import tpu as pltpu
from jax.experimental.pallas import tpu_sc as plsc

NUM_CORES = 2
NUM_SUBCORES = 16
NW = NUM_CORES * NUM_SUBCORES

D = 32
BBLK = 128
NBUF = 4


def _gather_body(textT_hbm, tbl_hbm, out_hbm, txt_all, sidx, g, outT,
                 gsem, osem):
    n = textT_hbm.shape[0]
    wid = lax.axis_index("s") * NUM_CORES + lax.axis_index("c")
    bg = wid
    b0 = bg * BBLK

    pltpu.sync_copy(textT_hbm.at[:, pl.ds(b0, BBLK)], txt_all)

    iota16 = lax.iota(jnp.int32, 16)
    zero16 = iota16 * 0

    def prep(l, p):
        for k in range(BBLK // 16):
            sidx[p, pl.ds(16 * k, 16)] = txt_all[l, pl.ds(16 * k, 16)]

    def start_gather(l, p):
        prep(l, p)
        pltpu.async_copy(tbl_hbm.at[sidx.at[p]], g.at[p], gsem.at[p])

    def wait_gather(p):
        pltpu.make_async_copy(tbl_hbm.at[sidx.at[p]], g.at[p],
                              gsem.at[p]).wait()

    def compact(p):
        gp = g.at[p]

        @plsc.parallel_loop(0, BBLK // 16, unroll=4)
        def _cp(tg):
            rows = iota16 + 16 * tg
            for d in range(D):
                vec = plsc.load_gather(gp, [rows, zero16 + d])
                outT[p, 0, d // 8, 0, d % 8, pl.ds(16 * tg, 16)] = vec

    def start_out(l, p):
        pltpu.async_copy(
            outT.at[p],
            out_hbm.at[pl.ds(l, 1), pl.ds(0, 4), pl.ds(bg, 1),
                       pl.ds(0, 8), pl.ds(0, BBLK)],
            osem.at[p])

    def wait_out(l, p):
        pltpu.make_async_copy(
            outT.at[p],
            out_hbm.at[pl.ds(l, 1), pl.ds(0, 4), pl.ds(bg, 1),
                       pl.ds(0, 8), pl.ds(0, BBLK)],
            osem.at[p]).wait()

    for b in range(NBUF - 1):
        start_gather(b, b)
    wait_gather(0)
    compact(0)
    start_out(0, 0)
    start_gather(NBUF - 1, NBUF - 1)

    def step(gi, carry):
        for b in range(NBUF):
            l = gi * NBUF + 1 + b
            p = (1 + b) % NBUF
            q = b % NBUF
            wait_gather(p)
            compact(p)
            start_out(l, p)
            wait_out(l - 1, q)
            start_gather(l + NBUF - 1, q)
        return carry

    lax.fori_loop(0, (n - NBUF) // NBUF, step, 0)

    for k in range(NBUF - 1):
        l = n - NBUF + 1 + k
        p = l % NBUF
        wait_gather(p)
        compact(p)
        start_out(l, p)
        wait_out(l - 1, (l - 1) % NBUF)
    wait_out(n - 1, (n - 1) % NBUF)


def _embedding_gather(textT, table, max_len, batch):
    mesh = plsc.VectorSubcoreMesh(core_axis_name="c", subcore_axis_name="s")
    grab = pl.kernel(
        _gather_body,
        out_type=jax.ShapeDtypeStruct((max_len, 4, batch // BBLK, 8, BBLK),
                                      jnp.float32),
        mesh=mesh,
        scratch_types=[
            pltpu.VMEM((max_len, BBLK), jnp.int32),
            pltpu.VMEM((NBUF, BBLK), jnp.int32),
            pltpu.VMEM((NBUF, BBLK, D), jnp.float32),
            pltpu.VMEM((NBUF, 1, 4, 1, 8, BBLK), jnp.float32),
            pltpu.SemaphoreType.DMA((NBUF,)),
            pltpu.SemaphoreType.DMA((NBUF,)),
        ],
        compiler_params=pltpu.CompilerParams(
            use_tc_tiling_on_sc=False, needs_layout_passes=False),
    )
    return grab(textT, table)


def kernel(text, label, table):
    batch, max_len = text.shape
    textT = jnp.transpose(text).astype(jnp.int32)
    out5 = _embedding_gather(textT, table, max_len, batch)
    emb = out5.transpose(2, 4, 0, 1, 3).reshape(batch, max_len, D)
    return emb, label


# speedup vs baseline: 1.1412x; 1.1412x over previous
import jax
import jax.numpy as jnp
from jax import lax
from jax.experimental import pallas as pl
from jax.experimental.pallas import tpu as pltpu
from jax.experimental.pallas import tpu_sc as plsc

NUM_CORES = 2
NUM_SUBCORES = 16
NW = NUM_CORES * NUM_SUBCORES

D = 32
BBLK = 128
NBUF = 4


def _gather_body(textT_hbm, tbl_hbm, out_hbm, txt_all, sidx, g, outT,
                 gsem, osem):
    n = textT_hbm.shape[0]
    wid = lax.axis_index("s") * NUM_CORES + lax.axis_index("c")
    bg = wid
    b0 = bg * BBLK

    pltpu.sync_copy(textT_hbm.at[:, pl.ds(b0, BBLK)], txt_all)

    iota16 = lax.iota(jnp.int32, 16)
    zero16 = iota16 * 0

    def prep(l, p):
        for k in range(BBLK // 16):
            sidx[p, pl.ds(16 * k, 16)] = txt_all[l, pl.ds(16 * k, 16)]

    def start_gather(l, p):
        prep(l, p)
        pltpu.async_copy(tbl_hbm.at[sidx.at[p]], g.at[p], gsem.at[p])

    def wait_gather(p):
        pltpu.make_async_copy(tbl_hbm.at[sidx.at[p]], g.at[p],
                              gsem.at[p]).wait()

    def compact(p):
        gp = g.at[p]

        @plsc.parallel_loop(0, (BBLK // 16) * D, unroll=32)
        def _cp(i):
            tg = lax.div(i, D)
            d = lax.rem(i, D)
            rows = iota16 + 16 * tg
            cols = zero16 + d
            vec = plsc.load_gather(gp, [rows, cols])
            outT[p, 0, lax.div(d, 8), 0, lax.rem(d, 8),
                 pl.ds(16 * tg, 16)] = vec

    def start_out(l, p):
        pltpu.async_copy(
            outT.at[p],
            out_hbm.at[pl.ds(l, 1), pl.ds(0, 4), pl.ds(bg, 1),
                       pl.ds(0, 8), pl.ds(0, BBLK)],
            osem.at[p])

    def wait_out(l, p):
        pltpu.make_async_copy(
            outT.at[p],
            out_hbm.at[pl.ds(l, 1), pl.ds(0, 4), pl.ds(bg, 1),
                       pl.ds(0, 8), pl.ds(0, BBLK)],
            osem.at[p]).wait()

    for b in range(NBUF - 1):
        start_gather(b, b)
    wait_gather(0)
    compact(0)
    start_out(0, 0)
    start_gather(NBUF - 1, NBUF - 1)

    def step(gi, carry):
        for b in range(NBUF):
            l = gi * NBUF + 1 + b
            p = (1 + b) % NBUF
            q = b % NBUF
            wait_gather(p)
            compact(p)
            start_out(l, p)
            wait_out(l - 1, q)
            start_gather(l + NBUF - 1, q)
        return carry

    lax.fori_loop(0, (n - NBUF) // NBUF, step, 0)

    for k in range(NBUF - 1):
        l = n - NBUF + 1 + k
        p = l % NBUF
        wait_gather(p)
        compact(p)
        start_out(l, p)
        wait_out(l - 1, (l - 1) % NBUF)
    wait_out(n - 1, (n - 1) % NBUF)


def _embedding_gather(textT, table, max_len, batch):
    mesh = plsc.VectorSubcoreMesh(core_axis_name="c", subcore_axis_name="s")
    grab = pl.kernel(
        _gather_body,
        out_type=jax.ShapeDtypeStruct((max_len, 4, batch // BBLK, 8, BBLK),
                                      jnp.float32),
        mesh=mesh,
        scratch_types=[
            pltpu.VMEM((max_len, BBLK), jnp.int32),
            pltpu.VMEM((NBUF, BBLK), jnp.int32),
            pltpu.VMEM((NBUF, BBLK, D), jnp.float32),
            pltpu.VMEM((NBUF, 1, 4, 1, 8, BBLK), jnp.float32),
            pltpu.SemaphoreType.DMA((NBUF,)),
            pltpu.SemaphoreType.DMA((NBUF,)),
        ],
        compiler_params=pltpu.CompilerParams(
            use_tc_tiling_on_sc=False, needs_layout_passes=False),
    )
    return grab(textT, table)


def kernel(text, label, table):
    batch, max_len = text.shape
    textT = jnp.transpose(text).astype(jnp.int32)
    out5 = _embedding_gather(textT, table, max_len, batch)
    emb = out5.transpose(2, 4, 0, 1, 3).reshape(batch, max_len, D)
    return emb, label
